# baseline (speedup 1.0000x reference)
import jax
import jax.numpy as jnp
from jax import lax
from jax.experimental import pallas as pl
from jax.experimental.pallas import tpu as pltpu
from jax.experimental.pallas import tpu_sc as plsc

B = 8
N = 4096
L = 16
R_SC = 0
NCHUNK = 4
ROWS = R_SC // NCHUNK
IB = 8
NJ = N // L
TR = 256
NTC = N - R_SC


def _round_bf16(v):
    r = lax.bitcast_convert_type(v, jnp.uint32)
    r = r + jnp.uint32(0x7FFF) + (lax.shift_right_logical(r, jnp.uint32(16))
                                  & jnp.uint32(1))
    r = r & jnp.uint32(0xFFFF0000)
    return lax.bitcast_convert_type(r, jnp.float32)


def _vsqrt(d):
    d = jnp.maximum(d, jnp.float32(1e-30))
    i = lax.bitcast_convert_type(d, jnp.int32)
    i = jnp.int32(0x5F3759DF) - lax.shift_right_arithmetic(i, jnp.int32(1))
    y = lax.bitcast_convert_type(i, jnp.float32)
    for _ in range(3):
        y = y * (jnp.float32(1.5) - jnp.float32(0.5) * d * y * y)
    return d * y


def _sc_body(x1_hbm, x2_hbm, out1_hbm, out2_hbm,
             x1v, x2v, asqv, bsqv, d2v, d1v, cmb, s1v, shared):
    c = lax.axis_index("c")
    s = lax.axis_index("s")
    batch = c * NCHUNK + s // NCHUNK
    chunk = s % NCHUNK

    pltpu.sync_copy(x2_hbm.at[batch], x2v)
    pltpu.sync_copy(x1_hbm.at[batch], x1v)
    coff = chunk * ROWS

    inf16 = jnp.full((L,), jnp.inf, jnp.float32)

    def init_b(j, _):
        sl = pl.ds(j * L, L)
        bsqv[sl] = x2v[3, sl] + x2v[4, sl]
        d2v[sl] = inf16
        return 0
    lax.fori_loop(0, NJ, init_b, 0)

    def init_a(i, _):
        sl = pl.ds(coff + i * L, L)
        fx = x1v[3, sl]; fy = x1v[4, sl]; fz = x1v[5, sl]
        asqv[pl.ds(i * L, L)] = (fx * fx + fy * fy) + fz * fz
        return 0
    lax.fori_loop(0, ROWS // L, init_a, 0)

    lane = lax.broadcasted_iota(jnp.int32, (L,), 0)

    def outer(ib, _):
        base = ib * L
        cx = x1v[0, pl.ds(coff + base, L)]
        cy = x1v[1, pl.ds(coff + base, L)]
        cz = x1v[2, pl.ds(coff + base, L)]
        cq = asqv[pl.ds(base, L)]
        row_mins = []
        for k0 in (0, IB):
            sx = [cx[k0 + k] for k in range(IB)]
            sy = [cy[k0 + k] for k in range(IB)]
            sz = [cz[k0 + k] for k in range(IB)]
            sq_ = [cq[k0 + k] for k in range(IB)]

            def inner(j, mins):
                sl = pl.ds(j * L, L)
                bx = x2v[0, sl]
                by = x2v[1, sl]
                bz = x2v[2, sl]
                bq = bsqv[sl]
                ds_ = []
                new_mins = []
                for k in range(IB):
                    pr = sx[k] * bx + sy[k] * by + sz[k] * bz
                    dd = (sq_[k] + bq) + pr
                    ds_.append(dd)
                    new_mins.append(jnp.minimum(mins[k], dd))
                t = ds_
                while len(t) > 1:
                    t = [jnp.minimum(t[2 * a], t[2 * a + 1])
                         for a in range(len(t) // 2)]
                d2v[sl] = jnp.minimum(d2v[sl], t[0])
                return tuple(new_mins)

            mins = lax.fori_loop(0, NJ, inner, (inf16,) * IB)
            row_mins.extend(jnp.min(m) for m in mins)
        vec = jnp.zeros((L,), jnp.float32)
        for k in range(L):
            vec = jnp.where(lane == k, row_mins[k], vec)
        d1v[pl.ds(base, L)] = vec
        return 0

    lax.fori_loop(0, ROWS // L, outer, 0)

    def sum1(j, acc):
        return acc + _vsqrt(d1v[pl.ds(j * L, L)])
    s1v[...] = lax.fori_loop(0, ROWS // L, sum1, jnp.zeros((L,), jnp.float32))
    pltpu.sync_copy(s1v, out1_hbm.at[c * 16 + s])

    pltpu.sync_copy(d2v, shared.at[s])
    plsc.subcore_barrier()

    @pl.when(chunk == 0)
    def _leader():
        for k in range(NCHUNK):
            pltpu.sync_copy(shared.at[s + k], cmb.at[k])

        def comb(j, _):
            sl = pl.ds(j * L, L)
            d2v[sl] = jnp.minimum(jnp.minimum(cmb[0, sl], cmb[1, sl]),
                                  jnp.minimum(cmb[2, sl], cmb[3, sl]))
            return 0
        lax.fori_loop(0, NJ, comb, 0)
        pltpu.sync_copy(d2v, out2_hbm.at[batch])


def _chamfer_sc(x1all, x2all):
    mesh = plsc.VectorSubcoreMesh(core_axis_name="c", subcore_axis_name="s",
                                  num_cores=2, num_subcores=16)
    run = pl.kernel(
        _sc_body,
        mesh=mesh,
        compiler_params=pltpu.CompilerParams(needs_layout_passes=False),
        out_type=(
            jax.ShapeDtypeStruct((2 * 16, L), jnp.float32),
            jax.ShapeDtypeStruct((B, N), jnp.float32),
        ),
        scratch_types=[
            pltpu.VMEM((6, R_SC), jnp.float32),
            pltpu.VMEM((5, N), jnp.float32),
            pltpu.VMEM((ROWS,), jnp.float32),
            pltpu.VMEM((N,), jnp.float32),
            pltpu.VMEM((N,), jnp.float32),
            pltpu.VMEM((ROWS,), jnp.float32),
            pltpu.VMEM((NCHUNK, N), jnp.float32),
            pltpu.VMEM((L,), jnp.float32),
            pltpu.VMEM_SHARED((16, N), jnp.float32),
        ],
    )
    return run(x1all, x2all)


def _pack_sc(xyz, scale):
    full = jnp.transpose(xyz, (0, 2, 1))
    rnd = _round_bf16(full) * jnp.float32(scale)
    return jnp.concatenate([rnd, full], axis=1)


def _tc_body_full(a_ref, b_ref, out1_ref, out2_ref):
    d2 = jnp.full((N,), jnp.inf, jnp.float32)
    d1sums = jnp.zeros((128,), jnp.float32)
    for r in range(N // TR):
        a = a_ref[0, r * TR:(r + 1) * TR, :]
        sq = jax.lax.dot_general(
            a, b_ref[0], (((1,), (1,)), ((), ())),
            preferred_element_type=jnp.float32)
        d1 = jnp.sqrt(jnp.maximum(jnp.min(sq, axis=1), 0.0))
        d1sums = d1sums + jnp.sum(d1.reshape(TR // 128, 128), axis=0)
        d2 = jnp.minimum(d2, jnp.min(sq, axis=0))
    out1_ref[0, 0, :] = d1sums
    d2s = jnp.sqrt(jnp.maximum(d2, 0.0))
    out2_ref[0, 0, :] = jnp.sum(d2s.reshape(N // 128, 128), axis=0)


def _chamfer_tc_full(a, b):
    return pl.pallas_call(
        _tc_body_full,
        grid=(B,),
        in_specs=[
            pl.BlockSpec((1, N, 8), lambda i: (i, 0, 0)),
            pl.BlockSpec((1, N, 8), lambda i: (i, 0, 0)),
        ],
        out_specs=[
            pl.BlockSpec((1, 1, 128), lambda i: (i, 0, 0)),
            pl.BlockSpec((1, 1, 128), lambda i: (i, 0, 0)),
        ],
        out_shape=[
            jax.ShapeDtypeStruct((B, 1, 128), jnp.float32),
            jax.ShapeDtypeStruct((B, 1, 128), jnp.float32),
        ],
    )(a, b)


def _tc_body(a_ref, b_ref, out1_ref, out2_ref):
    d2 = jnp.full((N,), jnp.inf, jnp.float32)
    d1sums = jnp.zeros((128,), jnp.float32)
    for r in range(NTC // TR):
        a = a_ref[0, r * TR:(r + 1) * TR, :]
        sq = jax.lax.dot_general(
            a, b_ref[0], (((1,), (0,)), ((), ())),
            preferred_element_type=jnp.float32)
        d1 = jnp.sqrt(jnp.maximum(jnp.min(sq, axis=1), 0.0))
        d1sums = d1sums + jnp.sum(d1.reshape(TR // 128, 128), axis=0)
        d2 = jnp.minimum(d2, jnp.min(sq, axis=0))
    out1_ref[0, 0, :] = d1sums
    out2_ref[0, 0, :] = d2


def _chamfer_tc(a, b):
    return pl.pallas_call(
        _tc_body,
        grid=(B,),
        in_specs=[
            pl.BlockSpec((1, NTC, 8), lambda i: (i, 0, 0)),
            pl.BlockSpec((1, 8, N), lambda i: (i, 0, 0)),
        ],
        out_specs=[
            pl.BlockSpec((1, 1, 128), lambda i: (i, 0, 0)),
            pl.BlockSpec((1, 1, N), lambda i: (i, 0, 0)),
        ],
        out_shape=[
            jax.ShapeDtypeStruct((B, 1, 128), jnp.float32),
            jax.ShapeDtypeStruct((B, 1, N), jnp.float32),
        ],
    )(a, b)


def _merge_body(sc_ref, tc_ref, out_ref):
    w = jnp.minimum(sc_ref[...], tc_ref[...])
    d2 = jnp.sqrt(jnp.maximum(w, 0.0))
    out_ref[...] = jnp.sum(d2.reshape(B, N // 128, 128), axis=1)


def _merge_d2(d2_sc, d2_tc):
    return pl.pallas_call(
        _merge_body,
        out_shape=jax.ShapeDtypeStruct((B, 128), jnp.float32),
    )(d2_sc, d2_tc)


def _aug1(xyz):
    r = _round_bf16(xyz) * jnp.float32(-2.0)
    x, y, z = xyz[..., 0], xyz[..., 1], xyz[..., 2]
    asq = (x * x + y * y) + z * z
    hi = _round_bf16(asq)
    lo = _round_bf16(asq - hi)
    one = jnp.ones_like(asq)
    zero = jnp.zeros_like(asq)
    return jnp.concatenate(
        [r] + [v[..., None] for v in (hi, lo, one, one, zero)], axis=2)


def _aug2(xyz):
    r = _round_bf16(xyz)
    x, y, z = xyz[..., 0], xyz[..., 1], xyz[..., 2]
    bsq = (x * x + y * y) + z * z
    hi = _round_bf16(bsq)
    lo = _round_bf16(bsq - hi)
    one = jnp.ones_like(bsq)
    zero = jnp.zeros_like(bsq)
    rx, ry, rz = r[..., 0], r[..., 1], r[..., 2]
    b_aug = jnp.concatenate(
        [r] + [v[..., None] for v in (one, one, hi, lo, zero)], axis=2)
    b_sc = jnp.stack([rx, ry, rz, hi, lo], axis=1)
    return b_aug, b_sc


def kernel(xyz1, xyz2):
    inv = jnp.float32(1.0 / (B * N))
    b_aug, b_sc = _aug2(xyz2)
    if R_SC:
        s1, d2_sc = _chamfer_sc(_pack_sc(xyz1[:, :R_SC], -2.0), b_sc)
        t1, d2_tc = _chamfer_tc(_aug1(xyz1[:, R_SC:]), b_aug)
        m2 = _merge_d2(d2_sc, d2_tc.reshape(B, N))
        return (s1.sum() + t1.sum() + m2.sum()) * inv
    t1, t2 = _chamfer_tc_full(_aug1(xyz1), b_aug)
    return (t1.sum() + t2.sum()) * inv


# speedup vs baseline: 3.9234x; 3.9234x over previous
import jax
import jax.numpy as jnp
from jax import lax
from jax.experimental import pallas as pl
from jax.experimental.pallas import tpu as pltpu
from jax.experimental.pallas import tpu_sc as plsc

B = 8
N = 4096
L = 16
R_SC = 0
NCHUNK = 4
ROWS = R_SC // NCHUNK
IB = 8
NJ = N // L
TR = 256
NTC = N - R_SC


def _round_bf16(v):
    r = lax.bitcast_convert_type(v, jnp.uint32)
    r = r + jnp.uint32(0x7FFF) + (lax.shift_right_logical(r, jnp.uint32(16))
                                  & jnp.uint32(1))
    r = r & jnp.uint32(0xFFFF0000)
    return lax.bitcast_convert_type(r, jnp.float32)


def _vsqrt(d):
    d = jnp.maximum(d, jnp.float32(1e-30))
    i = lax.bitcast_convert_type(d, jnp.int32)
    i = jnp.int32(0x5F3759DF) - lax.shift_right_arithmetic(i, jnp.int32(1))
    y = lax.bitcast_convert_type(i, jnp.float32)
    for _ in range(3):
        y = y * (jnp.float32(1.5) - jnp.float32(0.5) * d * y * y)
    return d * y


def _sc_body(x1_hbm, x2_hbm, out1_hbm, out2_hbm,
             x1v, x2v, asqv, bsqv, d2v, d1v, cmb, s1v, shared):
    c = lax.axis_index("c")
    s = lax.axis_index("s")
    batch = c * NCHUNK + s // NCHUNK
    chunk = s % NCHUNK

    pltpu.sync_copy(x2_hbm.at[batch], x2v)
    pltpu.sync_copy(x1_hbm.at[batch], x1v)
    coff = chunk * ROWS

    inf16 = jnp.full((L,), jnp.inf, jnp.float32)

    def init_b(j, _):
        sl = pl.ds(j * L, L)
        bsqv[sl] = x2v[3, sl] + x2v[4, sl]
        d2v[sl] = inf16
        return 0
    lax.fori_loop(0, NJ, init_b, 0)

    def init_a(i, _):
        sl = pl.ds(coff + i * L, L)
        fx = x1v[3, sl]; fy = x1v[4, sl]; fz = x1v[5, sl]
        asqv[pl.ds(i * L, L)] = (fx * fx + fy * fy) + fz * fz
        return 0
    lax.fori_loop(0, ROWS // L, init_a, 0)

    lane = lax.broadcasted_iota(jnp.int32, (L,), 0)

    def outer(ib, _):
        base = ib * L
        cx = x1v[0, pl.ds(coff + base, L)]
        cy = x1v[1, pl.ds(coff + base, L)]
        cz = x1v[2, pl.ds(coff + base, L)]
        cq = asqv[pl.ds(base, L)]
        row_mins = []
        for k0 in (0, IB):
            sx = [cx[k0 + k] for k in range(IB)]
            sy = [cy[k0 + k] for k in range(IB)]
            sz = [cz[k0 + k] for k in range(IB)]
            sq_ = [cq[k0 + k] for k in range(IB)]

            def inner(j, mins):
                sl = pl.ds(j * L, L)
                bx = x2v[0, sl]
                by = x2v[1, sl]
                bz = x2v[2, sl]
                bq = bsqv[sl]
                ds_ = []
                new_mins = []
                for k in range(IB):
                    pr = sx[k] * bx + sy[k] * by + sz[k] * bz
                    dd = (sq_[k] + bq) + pr
                    ds_.append(dd)
                    new_mins.append(jnp.minimum(mins[k], dd))
                t = ds_
                while len(t) > 1:
                    t = [jnp.minimum(t[2 * a], t[2 * a + 1])
                         for a in range(len(t) // 2)]
                d2v[sl] = jnp.minimum(d2v[sl], t[0])
                return tuple(new_mins)

            mins = lax.fori_loop(0, NJ, inner, (inf16,) * IB)
            row_mins.extend(jnp.min(m) for m in mins)
        vec = jnp.zeros((L,), jnp.float32)
        for k in range(L):
            vec = jnp.where(lane == k, row_mins[k], vec)
        d1v[pl.ds(base, L)] = vec
        return 0

    lax.fori_loop(0, ROWS // L, outer, 0)

    def sum1(j, acc):
        return acc + _vsqrt(d1v[pl.ds(j * L, L)])
    s1v[...] = lax.fori_loop(0, ROWS // L, sum1, jnp.zeros((L,), jnp.float32))
    pltpu.sync_copy(s1v, out1_hbm.at[c * 16 + s])

    pltpu.sync_copy(d2v, shared.at[s])
    plsc.subcore_barrier()

    @pl.when(chunk == 0)
    def _leader():
        for k in range(NCHUNK):
            pltpu.sync_copy(shared.at[s + k], cmb.at[k])

        def comb(j, _):
            sl = pl.ds(j * L, L)
            d2v[sl] = jnp.minimum(jnp.minimum(cmb[0, sl], cmb[1, sl]),
                                  jnp.minimum(cmb[2, sl], cmb[3, sl]))
            return 0
        lax.fori_loop(0, NJ, comb, 0)
        pltpu.sync_copy(d2v, out2_hbm.at[batch])


def _chamfer_sc(x1all, x2all):
    mesh = plsc.VectorSubcoreMesh(core_axis_name="c", subcore_axis_name="s",
                                  num_cores=2, num_subcores=16)
    run = pl.kernel(
        _sc_body,
        mesh=mesh,
        compiler_params=pltpu.CompilerParams(needs_layout_passes=False),
        out_type=(
            jax.ShapeDtypeStruct((2 * 16, L), jnp.float32),
            jax.ShapeDtypeStruct((B, N), jnp.float32),
        ),
        scratch_types=[
            pltpu.VMEM((6, R_SC), jnp.float32),
            pltpu.VMEM((5, N), jnp.float32),
            pltpu.VMEM((ROWS,), jnp.float32),
            pltpu.VMEM((N,), jnp.float32),
            pltpu.VMEM((N,), jnp.float32),
            pltpu.VMEM((ROWS,), jnp.float32),
            pltpu.VMEM((NCHUNK, N), jnp.float32),
            pltpu.VMEM((L,), jnp.float32),
            pltpu.VMEM_SHARED((16, N), jnp.float32),
        ],
    )
    return run(x1all, x2all)


def _pack_sc(xyz, scale):
    full = jnp.transpose(xyz, (0, 2, 1))
    rnd = _round_bf16(full) * jnp.float32(scale)
    return jnp.concatenate([rnd, full], axis=1)


def _tc_body_full(a_ref, b_ref, out1_ref, out2_ref):
    d2 = jnp.full((N,), jnp.inf, jnp.float32)
    d1sums = jnp.zeros((128,), jnp.float32)
    for r in range(N // TR):
        a = a_ref[0, r * TR:(r + 1) * TR, :]
        sq = jax.lax.dot_general(
            a, b_ref[0], (((1,), (0,)), ((), ())),
            preferred_element_type=jnp.float32)
        d1 = jnp.sqrt(jnp.maximum(jnp.min(sq, axis=1), 0.0))
        d1sums = d1sums + jnp.sum(d1.reshape(TR // 128, 128), axis=0)
        d2 = jnp.minimum(d2, jnp.min(sq, axis=0))
    out1_ref[0, 0, :] = d1sums
    d2s = jnp.sqrt(jnp.maximum(d2, 0.0))
    out2_ref[0, 0, :] = jnp.sum(d2s.reshape(N // 128, 128), axis=0)


def _chamfer_tc_full(a, b):
    return pl.pallas_call(
        _tc_body_full,
        grid=(B,),
        in_specs=[
            pl.BlockSpec((1, N, 8), lambda i: (i, 0, 0)),
            pl.BlockSpec((1, 8, N), lambda i: (i, 0, 0)),
        ],
        out_specs=[
            pl.BlockSpec((1, 1, 128), lambda i: (i, 0, 0)),
            pl.BlockSpec((1, 1, 128), lambda i: (i, 0, 0)),
        ],
        out_shape=[
            jax.ShapeDtypeStruct((B, 1, 128), jnp.float32),
            jax.ShapeDtypeStruct((B, 1, 128), jnp.float32),
        ],
    )(a, b)


def _tc_body(a_ref, b_ref, out1_ref, out2_ref):
    d2 = jnp.full((N,), jnp.inf, jnp.float32)
    d1sums = jnp.zeros((128,), jnp.float32)
    for r in range(NTC // TR):
        a = a_ref[0, r * TR:(r + 1) * TR, :]
        sq = jax.lax.dot_general(
            a, b_ref[0], (((1,), (0,)), ((), ())),
            preferred_element_type=jnp.float32)
        d1 = jnp.sqrt(jnp.maximum(jnp.min(sq, axis=1), 0.0))
        d1sums = d1sums + jnp.sum(d1.reshape(TR // 128, 128), axis=0)
        d2 = jnp.minimum(d2, jnp.min(sq, axis=0))
    out1_ref[0, 0, :] = d1sums
    out2_ref[0, 0, :] = d2


def _chamfer_tc(a, b):
    return pl.pallas_call(
        _tc_body,
        grid=(B,),
        in_specs=[
            pl.BlockSpec((1, NTC, 8), lambda i: (i, 0, 0)),
            pl.BlockSpec((1, 8, N), lambda i: (i, 0, 0)),
        ],
        out_specs=[
            pl.BlockSpec((1, 1, 128), lambda i: (i, 0, 0)),
            pl.BlockSpec((1, 1, N), lambda i: (i, 0, 0)),
        ],
        out_shape=[
            jax.ShapeDtypeStruct((B, 1, 128), jnp.float32),
            jax.ShapeDtypeStruct((B, 1, N), jnp.float32),
        ],
    )(a, b)


def _merge_body(sc_ref, tc_ref, out_ref):
    w = jnp.minimum(sc_ref[...], tc_ref[...])
    d2 = jnp.sqrt(jnp.maximum(w, 0.0))
    out_ref[...] = jnp.sum(d2.reshape(B, N // 128, 128), axis=1)


def _merge_d2(d2_sc, d2_tc):
    return pl.pallas_call(
        _merge_body,
        out_shape=jax.ShapeDtypeStruct((B, 128), jnp.float32),
    )(d2_sc, d2_tc)


def _aug1(xyz):
    r = _round_bf16(xyz) * jnp.float32(-2.0)
    x, y, z = xyz[..., 0], xyz[..., 1], xyz[..., 2]
    asq = (x * x + y * y) + z * z
    hi = _round_bf16(asq)
    lo = _round_bf16(asq - hi)
    one = jnp.ones_like(asq)
    zero = jnp.zeros_like(asq)
    return jnp.concatenate(
        [r] + [v[..., None] for v in (hi, lo, one, one, zero)], axis=2)


def _aug2(xyz):
    r = _round_bf16(xyz)
    x, y, z = xyz[..., 0], xyz[..., 1], xyz[..., 2]
    bsq = (x * x + y * y) + z * z
    hi = _round_bf16(bsq)
    lo = _round_bf16(bsq - hi)
    one = jnp.ones_like(bsq)
    zero = jnp.zeros_like(bsq)
    rx, ry, rz = r[..., 0], r[..., 1], r[..., 2]
    b_aug = jnp.stack([rx, ry, rz, one, one, hi, lo, zero], axis=1)
    b_sc = jnp.stack([rx, ry, rz, hi, lo], axis=1)
    return b_aug, b_sc


def kernel(xyz1, xyz2):
    inv = jnp.float32(1.0 / (B * N))
    b_aug, b_sc = _aug2(xyz2)
    if R_SC:
        s1, d2_sc = _chamfer_sc(_pack_sc(xyz1[:, :R_SC], -2.0), b_sc)
        t1, d2_tc = _chamfer_tc(_aug1(xyz1[:, R_SC:]), b_aug)
        m2 = _merge_d2(d2_sc, d2_tc.reshape(B, N))
        return (s1.sum() + t1.sum() + m2.sum()) * inv
    t1, t2 = _chamfer_tc_full(_aug1(xyz1), b_aug)
    return (t1.sum() + t2.sum()) * inv


# speedup vs baseline: 3.9875x; 1.0164x over previous
import jax
import jax.numpy as jnp
from jax import lax
from jax.experimental import pallas as pl
from jax.experimental.pallas import tpu as pltpu
from jax.experimental.pallas import tpu_sc as plsc

B = 8
N = 4096
L = 16
R_SC = 0
NCHUNK = 4
ROWS = R_SC // NCHUNK
IB = 8
NJ = N // L
TR = 256
NTC = N - R_SC


def _round_bf16(v):
    r = lax.bitcast_convert_type(v, jnp.uint32)
    r = r + jnp.uint32(0x7FFF) + (lax.shift_right_logical(r, jnp.uint32(16))
                                  & jnp.uint32(1))
    r = r & jnp.uint32(0xFFFF0000)
    return lax.bitcast_convert_type(r, jnp.float32)


def _vsqrt(d):
    d = jnp.maximum(d, jnp.float32(1e-30))
    i = lax.bitcast_convert_type(d, jnp.int32)
    i = jnp.int32(0x5F3759DF) - lax.shift_right_arithmetic(i, jnp.int32(1))
    y = lax.bitcast_convert_type(i, jnp.float32)
    for _ in range(3):
        y = y * (jnp.float32(1.5) - jnp.float32(0.5) * d * y * y)
    return d * y


def _sc_body(x1_hbm, x2_hbm, out1_hbm, out2_hbm,
             x1v, x2v, asqv, bsqv, d2v, d1v, cmb, s1v, shared):
    c = lax.axis_index("c")
    s = lax.axis_index("s")
    batch = c * NCHUNK + s // NCHUNK
    chunk = s % NCHUNK

    pltpu.sync_copy(x2_hbm.at[batch], x2v)
    pltpu.sync_copy(x1_hbm.at[batch], x1v)
    coff = chunk * ROWS

    inf16 = jnp.full((L,), jnp.inf, jnp.float32)

    def init_b(j, _):
        sl = pl.ds(j * L, L)
        bsqv[sl] = x2v[3, sl] + x2v[4, sl]
        d2v[sl] = inf16
        return 0
    lax.fori_loop(0, NJ, init_b, 0)

    def init_a(i, _):
        sl = pl.ds(coff + i * L, L)
        fx = x1v[3, sl]; fy = x1v[4, sl]; fz = x1v[5, sl]
        asqv[pl.ds(i * L, L)] = (fx * fx + fy * fy) + fz * fz
        return 0
    lax.fori_loop(0, ROWS // L, init_a, 0)

    lane = lax.broadcasted_iota(jnp.int32, (L,), 0)

    def outer(ib, _):
        base = ib * L
        cx = x1v[0, pl.ds(coff + base, L)]
        cy = x1v[1, pl.ds(coff + base, L)]
        cz = x1v[2, pl.ds(coff + base, L)]
        cq = asqv[pl.ds(base, L)]
        row_mins = []
        for k0 in (0, IB):
            sx = [cx[k0 + k] for k in range(IB)]
            sy = [cy[k0 + k] for k in range(IB)]
            sz = [cz[k0 + k] for k in range(IB)]
            sq_ = [cq[k0 + k] for k in range(IB)]

            def inner(j, mins):
                sl = pl.ds(j * L, L)
                bx = x2v[0, sl]
                by = x2v[1, sl]
                bz = x2v[2, sl]
                bq = bsqv[sl]
                ds_ = []
                new_mins = []
                for k in range(IB):
                    pr = sx[k] * bx + sy[k] * by + sz[k] * bz
                    dd = (sq_[k] + bq) + pr
                    ds_.append(dd)
                    new_mins.append(jnp.minimum(mins[k], dd))
                t = ds_
                while len(t) > 1:
                    t = [jnp.minimum(t[2 * a], t[2 * a + 1])
                         for a in range(len(t) // 2)]
                d2v[sl] = jnp.minimum(d2v[sl], t[0])
                return tuple(new_mins)

            mins = lax.fori_loop(0, NJ, inner, (inf16,) * IB)
            row_mins.extend(jnp.min(m) for m in mins)
        vec = jnp.zeros((L,), jnp.float32)
        for k in range(L):
            vec = jnp.where(lane == k, row_mins[k], vec)
        d1v[pl.ds(base, L)] = vec
        return 0

    lax.fori_loop(0, ROWS // L, outer, 0)

    def sum1(j, acc):
        return acc + _vsqrt(d1v[pl.ds(j * L, L)])
    s1v[...] = lax.fori_loop(0, ROWS // L, sum1, jnp.zeros((L,), jnp.float32))
    pltpu.sync_copy(s1v, out1_hbm.at[c * 16 + s])

    pltpu.sync_copy(d2v, shared.at[s])
    plsc.subcore_barrier()

    @pl.when(chunk == 0)
    def _leader():
        for k in range(NCHUNK):
            pltpu.sync_copy(shared.at[s + k], cmb.at[k])

        def comb(j, _):
            sl = pl.ds(j * L, L)
            d2v[sl] = jnp.minimum(jnp.minimum(cmb[0, sl], cmb[1, sl]),
                                  jnp.minimum(cmb[2, sl], cmb[3, sl]))
            return 0
        lax.fori_loop(0, NJ, comb, 0)
        pltpu.sync_copy(d2v, out2_hbm.at[batch])


def _chamfer_sc(x1all, x2all):
    mesh = plsc.VectorSubcoreMesh(core_axis_name="c", subcore_axis_name="s",
                                  num_cores=2, num_subcores=16)
    run = pl.kernel(
        _sc_body,
        mesh=mesh,
        compiler_params=pltpu.CompilerParams(needs_layout_passes=False),
        out_type=(
            jax.ShapeDtypeStruct((2 * 16, L), jnp.float32),
            jax.ShapeDtypeStruct((B, N), jnp.float32),
        ),
        scratch_types=[
            pltpu.VMEM((6, R_SC), jnp.float32),
            pltpu.VMEM((5, N), jnp.float32),
            pltpu.VMEM((ROWS,), jnp.float32),
            pltpu.VMEM((N,), jnp.float32),
            pltpu.VMEM((N,), jnp.float32),
            pltpu.VMEM((ROWS,), jnp.float32),
            pltpu.VMEM((NCHUNK, N), jnp.float32),
            pltpu.VMEM((L,), jnp.float32),
            pltpu.VMEM_SHARED((16, N), jnp.float32),
        ],
    )
    return run(x1all, x2all)


def _pack_sc(xyz, scale):
    full = jnp.transpose(xyz, (0, 2, 1))
    rnd = _round_bf16(full) * jnp.float32(scale)
    return jnp.concatenate([rnd, full], axis=1)


def _tc_body_full(a_ref, b_ref, out_ref):
    d2 = jnp.full((N,), jnp.inf, jnp.float32)
    d1sums = jnp.zeros((128,), jnp.float32)
    for r in range(N // TR):
        a = a_ref[0, r * TR:(r + 1) * TR, :]
        sq = jax.lax.dot_general(
            a, b_ref[0], (((1,), (0,)), ((), ())),
            preferred_element_type=jnp.float32)
        d1 = jnp.sqrt(jnp.maximum(jnp.min(sq, axis=1), 0.0))
        d1sums = d1sums + jnp.sum(d1.reshape(TR // 128, 128), axis=0)
        d2 = jnp.minimum(d2, jnp.min(sq, axis=0))
    d2s = jnp.sqrt(jnp.maximum(d2, 0.0))
    out_ref[0, 0, :] = d1sums + jnp.sum(d2s.reshape(N // 128, 128), axis=0)


def _chamfer_tc_full(a, b):
    return pl.pallas_call(
        _tc_body_full,
        grid=(B,),
        in_specs=[
            pl.BlockSpec((1, N, 8), lambda i: (i, 0, 0)),
            pl.BlockSpec((1, 8, N), lambda i: (i, 0, 0)),
        ],
        out_specs=pl.BlockSpec((1, 1, 128), lambda i: (i, 0, 0)),
        out_shape=jax.ShapeDtypeStruct((B, 1, 128), jnp.float32),
    )(a, b)


def _tc_body(a_ref, b_ref, out1_ref, out2_ref):
    d2 = jnp.full((N,), jnp.inf, jnp.float32)
    d1sums = jnp.zeros((128,), jnp.float32)
    for r in range(NTC // TR):
        a = a_ref[0, r * TR:(r + 1) * TR, :]
        sq = jax.lax.dot_general(
            a, b_ref[0], (((1,), (0,)), ((), ())),
            preferred_element_type=jnp.float32)
        d1 = jnp.sqrt(jnp.maximum(jnp.min(sq, axis=1), 0.0))
        d1sums = d1sums + jnp.sum(d1.reshape(TR // 128, 128), axis=0)
        d2 = jnp.minimum(d2, jnp.min(sq, axis=0))
    out1_ref[0, 0, :] = d1sums
    out2_ref[0, 0, :] = d2


def _chamfer_tc(a, b):
    return pl.pallas_call(
        _tc_body,
        grid=(B,),
        in_specs=[
            pl.BlockSpec((1, NTC, 8), lambda i: (i, 0, 0)),
            pl.BlockSpec((1, 8, N), lambda i: (i, 0, 0)),
        ],
        out_specs=[
            pl.BlockSpec((1, 1, 128), lambda i: (i, 0, 0)),
            pl.BlockSpec((1, 1, N), lambda i: (i, 0, 0)),
        ],
        out_shape=[
            jax.ShapeDtypeStruct((B, 1, 128), jnp.float32),
            jax.ShapeDtypeStruct((B, 1, N), jnp.float32),
        ],
    )(a, b)


def _merge_body(sc_ref, tc_ref, out_ref):
    w = jnp.minimum(sc_ref[...], tc_ref[...])
    d2 = jnp.sqrt(jnp.maximum(w, 0.0))
    out_ref[...] = jnp.sum(d2.reshape(B, N // 128, 128), axis=1)


def _merge_d2(d2_sc, d2_tc):
    return pl.pallas_call(
        _merge_body,
        out_shape=jax.ShapeDtypeStruct((B, 128), jnp.float32),
    )(d2_sc, d2_tc)


def _aug1(xyz):
    r = _round_bf16(xyz) * jnp.float32(-2.0)
    x, y, z = xyz[..., 0], xyz[..., 1], xyz[..., 2]
    asq = (x * x + y * y) + z * z
    hi = _round_bf16(asq)
    lo = _round_bf16(asq - hi)
    one = jnp.ones_like(asq)
    zero = jnp.zeros_like(asq)
    return jnp.concatenate(
        [r] + [v[..., None] for v in (hi, lo, one, one, zero)], axis=2)


def _aug2(xyz):
    r = _round_bf16(xyz)
    x, y, z = xyz[..., 0], xyz[..., 1], xyz[..., 2]
    bsq = (x * x + y * y) + z * z
    hi = _round_bf16(bsq)
    lo = _round_bf16(bsq - hi)
    one = jnp.ones_like(bsq)
    zero = jnp.zeros_like(bsq)
    rx, ry, rz = r[..., 0], r[..., 1], r[..., 2]
    b_aug = jnp.stack([rx, ry, rz, one, one, hi, lo, zero], axis=1)
    b_sc = jnp.stack([rx, ry, rz, hi, lo], axis=1)
    return b_aug, b_sc


def kernel(xyz1, xyz2):
    inv = jnp.float32(1.0 / (B * N))
    b_aug, b_sc = _aug2(xyz2)
    if R_SC:
        s1, d2_sc = _chamfer_sc(_pack_sc(xyz1[:, :R_SC], -2.0), b_sc)
        t1, d2_tc = _chamfer_tc(_aug1(xyz1[:, R_SC:]), b_aug)
        m2 = _merge_d2(d2_sc, d2_tc.reshape(B, N))
        return (s1.sum() + t1.sum() + m2.sum()) * inv
    t = _chamfer_tc_full(_aug1(xyz1), b_aug)
    return t.sum() * inv
